# baseline (speedup 1.0000x reference)
import functools

import jax
import jax.numpy as jnp
from jax import lax
from jax.experimental import pallas as pl
from jax.experimental.pallas import tpu as pltpu
from jax.experimental.pallas import tpu_sc as plsc

_BATCH = 16384
_EMB = 64
_NODES = 1000000
_CN = 256
_NCHUNK = -(-_NODES // _CN)
_TAIL_C = _NCHUNK - 1
_TAIL_BASE = _TAIL_C * _CN
_TAIL_W = _NODES - _TAIL_BASE
_SH1 = _CN.bit_length() - 1
_SH2 = _SH1 + 5

_NUM_CORES = 2
_NUM_SUBCORES = 16
_NUM_WORKERS = _NUM_CORES * _NUM_SUBCORES
_LANES = 16
_CL_MAX = -(-_NCHUNK // _NUM_WORKERS)
_NPAIR = (_CL_MAX + 1) // 2

_PIECE = 4096

_mesh = plsc.VectorSubcoreMesh(
    core_axis_name="c",
    subcore_axis_name="s",
    num_cores=_NUM_CORES,
    num_subcores=_NUM_SUBCORES,
)


@functools.partial(
    pl.kernel,
    out_type=jax.ShapeDtypeStruct((_BATCH, _EMB), jnp.float32),
    mesh=_mesh,
    scratch_types=[
        pltpu.VMEM((_PIECE,), jnp.int32),
        pltpu.VMEM((_BATCH,), jnp.int32),
        [pltpu.VMEM((_EMB, _CN), jnp.float32) for _ in range(2)],
        pltpu.VMEM((_EMB, _TAIL_W), jnp.float32),
        [pltpu.VMEM((1, _EMB), jnp.float32) for _ in range(_LANES)],
        [pltpu.SemaphoreType.DMA for _ in range(2)],
        pltpu.SemaphoreType.DMA,
    ],
    compiler_params=pltpu.CompilerParams(needs_layout_passes=False),
)
def _sc_gather(table_hbm, tail_hbm, idx_hbm, out_hbm, piece_v, list_v,
               bufs, tail_v, rowbufs, sems, rsem):
    wid = lax.axis_index("s") * _NUM_CORES + lax.axis_index("c")

    pltpu.sync_copy(tail_hbm, tail_v)

    cnt = jnp.int32(0)
    for p in range(_BATCH // _PIECE):
        pltpu.sync_copy(idx_hbm.at[pl.ds(p * _PIECE, _PIECE)], piece_v)

        def scan_group(g, off, p=p):
            gbase = pl.multiple_of(g * _LANES, _LANES)
            i16 = piece_v[pl.ds(gbase, _LANES)]
            k16 = lax.iota(jnp.int32, _LANES) + (p * _PIECE) + gbase
            mask = lax.shift_right_logical(i16, _SH1) % _NUM_WORKERS == wid
            m32 = mask.astype(jnp.int32)
            pos16 = off + plsc.cumsum(m32) - m32
            cl16 = lax.shift_right_logical(i16, _SH2)
            e16 = (cl16 << 23) | (k16 << 9) | (i16 & (_CN - 1))
            plsc.store_scatter(list_v, [pos16], e16, mask=mask)
            return off + plsc.all_reduce_population_count(mask)[0]

        cnt = lax.fori_loop(0, _PIECE // _LANES, scan_group, cnt,
                            unroll=False)

    nvreg = (cnt + _LANES - 1) // _LANES

    def fetch(cl, buf, sem):
        c = cl * _NUM_WORKERS + wid

        @pl.when(c < _TAIL_C)
        def _():
            pltpu.async_copy(
                table_hbm.at[:, pl.ds(pl.multiple_of(c * _CN, _CN), _CN)],
                buf, sem)

    def wait(cl, buf, sem):
        c = cl * _NUM_WORKERS + wid

        @pl.when(c < _TAIL_C)
        def _():
            pltpu.make_async_copy(
                table_hbm.at[:, pl.ds(pl.multiple_of(c * _CN, _CN), _CN)],
                buf, sem).wait()

    def serve(cl, src):
        def vreg_body(v, _):
            vbase = v * _LANES
            e16 = list_v[pl.ds(vbase, _LANES)]
            lane_ok = (lax.iota(jnp.int32, _LANES) + vbase) < cnt
            hit = jnp.logical_and(
                lax.shift_right_logical(e16, 23) == cl, lane_ok)
            h32 = hit.astype(jnp.int32)
            npop = plsc.all_reduce_population_count(hit)
            k16 = lax.shift_right_logical(e16, 9) & (_BATCH - 1)

            @pl.when(npop[0] > 0)
            def _():
                for j in range(_LANES):
                    @pl.when(h32[j] == 1)
                    def _(j=j):
                        ci = jnp.full((_LANES,), e16[j] & (_CN - 1),
                                      jnp.int32)
                        for a in range(_EMB // _LANES):
                            e_r = lax.iota(jnp.int32, _LANES) + a * _LANES
                            rowbufs[j][0, pl.ds(a * _LANES, _LANES)] = (
                                plsc.load_gather(src, [e_r, ci]))
                        pltpu.async_copy(
                            rowbufs[j], out_hbm.at[pl.ds(k16[j], 1)], rsem)
                for j in range(_LANES):
                    @pl.when(h32[j] == 1)
                    def _(j=j):
                        pltpu.make_async_copy(
                            rowbufs[0], out_hbm.at[pl.ds(0, 1)], rsem
                        ).wait()
            return ()

        lax.fori_loop(0, nvreg, vreg_body, (), unroll=False)

    def pair_body(pr, _):
        cl0 = 2 * pr
        fetch(cl0 + 1, bufs[1], sems[1])
        wait(cl0, bufs[0], sems[0])

        @pl.when((cl0 * _NUM_WORKERS + wid) < _TAIL_C)
        def _():
            serve(cl0, bufs[0])

        @pl.when((cl0 * _NUM_WORKERS + wid) == _TAIL_C)
        def _():
            serve(cl0, tail_v)

        fetch(cl0 + 2, bufs[0], sems[0])
        wait(cl0 + 1, bufs[1], sems[1])

        @pl.when(((cl0 + 1) * _NUM_WORKERS + wid) < _TAIL_C)
        def _():
            serve(cl0 + 1, bufs[1])

        @pl.when(((cl0 + 1) * _NUM_WORKERS + wid) == _TAIL_C)
        def _():
            serve(cl0 + 1, tail_v)

        return ()

    fetch(0, bufs[0], sems[0])
    lax.fori_loop(0, _NPAIR, pair_body, (), unroll=False)


@jax.jit
def kernel(idx, emb_weight):
    return _sc_gather(emb_weight.T, emb_weight[_TAIL_BASE:].T,
                      idx.astype(jnp.int32))


# speedup vs baseline: 1.0091x; 1.0091x over previous
import functools

import jax
import jax.numpy as jnp
from jax import lax
from jax.experimental import pallas as pl
from jax.experimental.pallas import tpu as pltpu
from jax.experimental.pallas import tpu_sc as plsc

_BATCH = 16384
_EMB = 64
_NODES = 1000000
_CN = 256
_NCHUNK = -(-_NODES // _CN)
_TAIL_C = _NCHUNK - 1
_TAIL_BASE = _TAIL_C * _CN
_TAIL_W = _NODES - _TAIL_BASE
_SH1 = _CN.bit_length() - 1
_SH2 = _SH1 + 5

_NUM_CORES = 2
_NUM_SUBCORES = 16
_NUM_WORKERS = _NUM_CORES * _NUM_SUBCORES
_LANES = 16
_CL_MAX = -(-_NCHUNK // _NUM_WORKERS)
_NPAIR = (_CL_MAX + 1) // 2

_PIECE = 4096

_mesh = plsc.VectorSubcoreMesh(
    core_axis_name="c",
    subcore_axis_name="s",
    num_cores=_NUM_CORES,
    num_subcores=_NUM_SUBCORES,
)


@functools.partial(
    pl.kernel,
    out_type=jax.ShapeDtypeStruct((_BATCH, _EMB), jnp.float32),
    mesh=_mesh,
    scratch_types=[
        pltpu.VMEM((_PIECE,), jnp.int32),
        pltpu.VMEM((_BATCH,), jnp.int32),
        [pltpu.VMEM((_EMB, _CN), jnp.float32) for _ in range(6)],
        pltpu.VMEM((_EMB, _TAIL_W), jnp.float32),
        [pltpu.VMEM((1, _EMB), jnp.float32) for _ in range(_LANES)],
        [pltpu.SemaphoreType.DMA for _ in range(6)],
        pltpu.SemaphoreType.DMA,
    ],
    compiler_params=pltpu.CompilerParams(needs_layout_passes=False),
)
def _sc_gather(table_hbm, tail_hbm, idx_hbm, out_hbm, piece_v, list_v,
               bufs, tail_v, rowbufs, sems, rsem):
    wid = lax.axis_index("s") * _NUM_CORES + lax.axis_index("c")

    pltpu.sync_copy(tail_hbm, tail_v)

    cnt = jnp.int32(0)
    for p in range(_BATCH // _PIECE):
        pltpu.sync_copy(idx_hbm.at[pl.ds(p * _PIECE, _PIECE)], piece_v)

        def scan_group(g, off, p=p):
            gbase = pl.multiple_of(g * _LANES, _LANES)
            i16 = piece_v[pl.ds(gbase, _LANES)]
            k16 = lax.iota(jnp.int32, _LANES) + (p * _PIECE) + gbase
            mask = lax.shift_right_logical(i16, _SH1) % _NUM_WORKERS == wid
            m32 = mask.astype(jnp.int32)
            pos16 = off + plsc.cumsum(m32) - m32
            cl16 = lax.shift_right_logical(i16, _SH2)
            e16 = (cl16 << 23) | (k16 << 9) | (i16 & (_CN - 1))
            plsc.store_scatter(list_v, [pos16], e16, mask=mask)
            return off + plsc.all_reduce_population_count(mask)[0]

        cnt = lax.fori_loop(0, _PIECE // _LANES, scan_group, cnt,
                            unroll=False)

    nvreg = (cnt + _LANES - 1) // _LANES

    def fetch(cl, buf, sem):
        c = cl * _NUM_WORKERS + wid

        @pl.when(c < _TAIL_C)
        def _():
            pltpu.async_copy(
                table_hbm.at[:, pl.ds(pl.multiple_of(c * _CN, _CN), _CN)],
                buf, sem)

    def wait(cl, buf, sem):
        c = cl * _NUM_WORKERS + wid

        @pl.when(c < _TAIL_C)
        def _():
            pltpu.make_async_copy(
                table_hbm.at[:, pl.ds(pl.multiple_of(c * _CN, _CN), _CN)],
                buf, sem).wait()

    def hits_of(e16, lane_ok, cl):
        hit = jnp.logical_and(
            lax.shift_right_logical(e16, 23) == cl, lane_ok)
        return hit.astype(jnp.int32), plsc.all_reduce_population_count(hit)

    def emit(j, e16, k16, src):
        ci = jnp.full((_LANES,), e16[j] & (_CN - 1), jnp.int32)
        for a in range(_EMB // _LANES):
            e_r = lax.iota(jnp.int32, _LANES) + a * _LANES
            rowbufs[j][0, pl.ds(a * _LANES, _LANES)] = (
                plsc.load_gather(src, [e_r, ci]))
        pltpu.async_copy(rowbufs[j], out_hbm.at[pl.ds(k16[j], 1)], rsem)

    def serve_one(e16, k16, h32, npop, src):
        @pl.when(npop[0] > 0)
        def _():
            for j in range(_LANES):
                @pl.when(h32[j] == 1)
                def _(j=j):
                    emit(j, e16, k16, src)
            for j in range(_LANES):
                @pl.when(h32[j] == 1)
                def _(j=j):
                    pltpu.make_async_copy(
                        rowbufs[0], out_hbm.at[pl.ds(0, 1)], rsem).wait()

    def serve3(g, bset):
        cl0 = 3 * g

        def vreg_body(v, _):
            vbase = v * _LANES
            e16 = list_v[pl.ds(vbase, _LANES)]
            lane_ok = (lax.iota(jnp.int32, _LANES) + vbase) < cnt
            k16 = lax.shift_right_logical(e16, 9) & (_BATCH - 1)
            for d in range(3):
                c = (cl0 + d) * _NUM_WORKERS + wid
                h32, npop = hits_of(e16, lane_ok, cl0 + d)

                @pl.when(c < _TAIL_C)
                def _(d=d, h32=h32, npop=npop):
                    serve_one(e16, k16, h32, npop, bset[d])
            return ()

        lax.fori_loop(0, nvreg, vreg_body, (), unroll=False)

    def serve_tail():
        cl_t = (_TAIL_C - (_TAIL_C % _NUM_WORKERS)) // _NUM_WORKERS

        def vreg_body(v, _):
            vbase = v * _LANES
            e16 = list_v[pl.ds(vbase, _LANES)]
            lane_ok = (lax.iota(jnp.int32, _LANES) + vbase) < cnt
            k16 = lax.shift_right_logical(e16, 9) & (_BATCH - 1)
            h32, npop = hits_of(e16, lane_ok, cl_t)
            serve_one(e16, k16, h32, npop, tail_v)
            return ()

        lax.fori_loop(0, nvreg, vreg_body, (), unroll=False)

    def fetch_group(g, bset, ssset):
        for d in range(3):
            fetch(3 * g + d, bset[d], ssset[d])

    def wait_group(g, bset, ssset):
        for d in range(3):
            wait(3 * g + d, bset[d], ssset[d])

    _NGRP_PAIR = (-(-_CL_MAX // 3) + 1) // 2

    def pair_body(pr, _):
        g0 = 2 * pr
        fetch_group(g0 + 1, bufs[3:6], sems[3:6])
        wait_group(g0, bufs[0:3], sems[0:3])
        serve3(g0, bufs[0:3])
        fetch_group(g0 + 2, bufs[0:3], sems[0:3])
        wait_group(g0 + 1, bufs[3:6], sems[3:6])
        serve3(g0 + 1, bufs[3:6])
        return ()

    fetch_group(0, bufs[0:3], sems[0:3])
    lax.fori_loop(0, _NGRP_PAIR, pair_body, (), unroll=False)

    @pl.when(wid == _TAIL_C % _NUM_WORKERS)
    def _():
        serve_tail()


@jax.jit
def kernel(idx, emb_weight):
    return _sc_gather(emb_weight.T, emb_weight[_TAIL_BASE:].T,
                      idx.astype(jnp.int32))
